# baseline (speedup 1.0000x reference)
import jax
import jax.numpy as jnp
from jax import lax
from jax.experimental import pallas as pl
from jax.experimental.pallas import tpu as pltpu
from jax.experimental.pallas import tpu_sc as plsc

R = 128
C = 100000
TEMPERATURE = 1.0
EPS = 1e-20

NC, NS = 2, 16
NW = NC * NS
ROWS_PER_W = R // NW

CW = 50000
NCH = C // CW
ZW = 20000
NZ = C // ZW

_NOISE = None


def _noise():
    global _NOISE
    if _NOISE is None:
        with jax.ensure_compile_time_eval():
            u = jax.random.uniform(jax.random.key(42), (R, C),
                                   dtype=jnp.float32, minval=0.0, maxval=1.0)
            g = -jnp.log(-jnp.log(u + EPS) + EPS)
            _NOISE = jax.block_until_ready(g.reshape(-1))
    return _NOISE


def _sc_body(x_hbm, g_hbm, o_hbm, xbuf, gbuf, zbuf):
    cid = lax.axis_index("c")
    sid = lax.axis_index("s")
    wid = sid * NC + cid
    lane = lax.iota(jnp.int32, 16)
    zero16 = jnp.zeros((16,), jnp.float32)

    def zfill(i, _):
        zbuf[pl.ds(i * 16, 16)] = zero16
        return 0
    lax.fori_loop(0, ZW // 16, zfill, 0)

    def row_body(r, _):
        row = wid * ROWS_PER_W + r
        base = row * C

        def chunk_body(ch, carry):
            m, si = carry
            off = base + ch * CW
            pltpu.sync_copy(x_hbm.at[pl.ds(off, CW)], xbuf)
            pltpu.sync_copy(g_hbm.at[pl.ds(off, CW)], gbuf)
            step0 = ch * (CW // 16)

            def vec_body(i, c2):
                m2, si2 = c2
                y = xbuf[pl.ds(i * 16, 16)] + gbuf[pl.ds(i * 16, 16)]
                upd = y > m2
                m2 = jnp.where(upd, y, m2)
                si2 = jnp.where(upd, step0 + i, si2)
                return (m2, si2)

            return lax.fori_loop(0, CW // 16, vec_body, (m, si))

        m0 = jnp.full((16,), -jnp.inf, jnp.float32)
        s0 = jnp.zeros((16,), jnp.int32)
        m, si = lax.fori_loop(0, NCH, chunk_body, (m0, s0))

        gmax = jnp.max(m)
        cand = si * 16 + lane
        cand = jnp.where(m == gmax, cand, jnp.int32(1 << 30))
        col = jnp.min(cand)

        def wchunk(z, _):
            off = z * ZW
            loc = col - off
            inb = (loc >= 0) & (loc < ZW)
            slot = (loc // 16) * 16
            hot = jnp.where(lane == loc - slot, jnp.float32(1.0), 0.0)

            @pl.when(inb)
            def _set():
                zbuf[pl.ds(slot, 16)] = hot

            pltpu.sync_copy(zbuf, o_hbm.at[pl.ds(base + off, ZW)])

            @pl.when(inb)
            def _clr():
                zbuf[pl.ds(slot, 16)] = zero16

            return 0

        lax.fori_loop(0, NZ, wchunk, 0)
        return 0

    lax.fori_loop(0, ROWS_PER_W, row_body, 0)


def _build(interpret=False):
    mesh = plsc.VectorSubcoreMesh(core_axis_name="c", subcore_axis_name="s",
                                  num_cores=NC, num_subcores=NS)
    return pl.kernel(
        _sc_body,
        out_type=jax.ShapeDtypeStruct((R * C,), jnp.float32),
        mesh=mesh,
        scratch_types=[
            pltpu.VMEM((CW,), jnp.float32),
            pltpu.VMEM((CW,), jnp.float32),
            pltpu.VMEM((ZW,), jnp.float32),
        ],
        compiler_params=pltpu.CompilerParams(needs_layout_passes=False),
        interpret=interpret,
    )


def kernel(input):
    g = _noise()
    flat = _build()(input.reshape(-1), g)
    return flat.reshape(R, C)


# speedup vs baseline: 1.2804x; 1.2804x over previous
import jax
import jax.numpy as jnp
from jax import lax
from jax.experimental import pallas as pl
from jax.experimental.pallas import tpu as pltpu
from jax.experimental.pallas import tpu_sc as plsc

R = 128
C = 100000
EPS = 1e-20

NC, NS = 2, 16
NW = NC * NS
ROWS_PER_W = R // NW

CB = 20000
NCH = C // CB
STEPS = CB // 16
U = 10
GROUPS = STEPS // U
ZW = 20000
NZ = C // ZW

_NOISE = None


def _noise():
    global _NOISE
    if _NOISE is None:
        with jax.ensure_compile_time_eval():
            u = jax.random.uniform(jax.random.key(42), (R, C),
                                   dtype=jnp.float32, minval=0.0, maxval=1.0)
            g = -jnp.log(-jnp.log(u + EPS) + EPS)
            _NOISE = jax.block_until_ready(g.reshape(-1))
    return _NOISE


def _sc_body(x_hbm, g_hbm, o_hbm,
             xb0, xb1, gb0, gb1, zbuf, pbuf,
             rs0, rs1, wsem, psem):
    cid = lax.axis_index("c")
    sid = lax.axis_index("s")
    wid = sid * NC + cid
    lane = lax.iota(jnp.int32, 16)
    zero16 = jnp.zeros((16,), jnp.float32)
    xbufs, gbufs, rsems = (xb0, xb1), (gb0, gb1), (rs0, rs1)

    def zfill(i, _):
        zbuf[pl.ds(i * 16, 16)] = zero16
        pbuf[pl.ds(i * 16, 16)] = zero16
        return 0
    lax.fori_loop(0, ZW // 16, zfill, 0)

    def issue_read(base, ch, slot):
        off = base + ch * CB
        pltpu.async_copy(x_hbm.at[pl.ds(off, CB)], xbufs[slot], rsems[slot])
        pltpu.async_copy(g_hbm.at[pl.ds(off, CB)], gbufs[slot], rsems[slot])

    def drain_read(base, ch, slot):
        off = base + ch * CB
        pltpu.make_async_copy(x_hbm.at[pl.ds(off, CB)], xbufs[slot],
                              rsems[slot]).wait()
        pltpu.make_async_copy(g_hbm.at[pl.ds(off, CB)], gbufs[slot],
                              rsems[slot]).wait()

    def row_body(r, ploc):
        row = wid * ROWS_PER_W + r
        base = row * C
        issue_read(base, 0, 0)

        ms = [jnp.full((16,), -jnp.inf, jnp.float32)] * U
        ss = [jnp.zeros((16,), jnp.int32)] * U
        carry = tuple(ms) + tuple(ss)
        for ch in range(NCH):
            slot = ch % 2
            if ch + 1 < NCH:
                issue_read(base, ch + 1, 1 - slot)
            drain_read(base, ch, slot)
            xb, gb = xbufs[slot], gbufs[slot]
            step0 = ch * STEPS

            def group(j, cr, xb=xb, gb=gb, step0=step0):
                cr = list(cr)
                for k in range(U):
                    o = (j * U + k) * 16
                    y = xb[pl.ds(o, 16)] + gb[pl.ds(o, 16)]
                    upd = y > cr[k]
                    cr[U + k] = jnp.where(upd, step0 + j * U + k, cr[U + k])
                    cr[k] = jnp.maximum(y, cr[k])
                return tuple(cr)

            carry = lax.fori_loop(0, GROUPS, group, carry)

        ms, ss = list(carry[:U]), list(carry[U:])
        M, S = ms[0], ss[0]
        for k in range(1, U):
            take = (ms[k] > M) | ((ms[k] == M) & (ss[k] < S))
            M = jnp.where(take, ms[k], M)
            S = jnp.where(take, ss[k], S)

        gmax = jnp.max(M)
        cand = S * 16 + lane
        cand = jnp.where(M == gmax, cand, jnp.int32(1 << 30))
        col = jnp.min(cand)

        hc = col // ZW
        loc = col - hc * ZW
        slot16 = (loc // 16) * 16

        @pl.when(r > 0)
        def _drain_patch():
            pltpu.make_async_copy(pbuf, o_hbm.at[pl.ds(0, ZW)], psem).wait()

        pbuf[pl.ds(ploc, 16)] = zero16
        pbuf[pl.ds(slot16, 16)] = jnp.where(lane == loc - slot16,
                                            jnp.float32(1.0), 0.0)
        for z in range(NZ):
            off = base + z * ZW
            hot = z == hc

            @pl.when(hot)
            def _wp(off=off):
                pltpu.async_copy(pbuf, o_hbm.at[pl.ds(off, ZW)], psem)

            @pl.when(jnp.logical_not(hot))
            def _wz(off=off):
                pltpu.async_copy(zbuf, o_hbm.at[pl.ds(off, ZW)], wsem)

        return slot16

    lax.fori_loop(0, ROWS_PER_W, row_body, jnp.int32(0))

    pltpu.make_async_copy(pbuf, o_hbm.at[pl.ds(0, ZW)], psem).wait()
    for _ in range(ROWS_PER_W * (NZ - 1)):
        pltpu.make_async_copy(zbuf, o_hbm.at[pl.ds(0, ZW)], wsem).wait()


def _build(interpret=False):
    mesh = plsc.VectorSubcoreMesh(core_axis_name="c", subcore_axis_name="s",
                                  num_cores=NC, num_subcores=NS)
    return pl.kernel(
        _sc_body,
        out_type=jax.ShapeDtypeStruct((R * C,), jnp.float32),
        mesh=mesh,
        scratch_types=[
            pltpu.VMEM((CB,), jnp.float32),
            pltpu.VMEM((CB,), jnp.float32),
            pltpu.VMEM((CB,), jnp.float32),
            pltpu.VMEM((CB,), jnp.float32),
            pltpu.VMEM((ZW,), jnp.float32),
            pltpu.VMEM((ZW,), jnp.float32),
            pltpu.SemaphoreType.DMA,
            pltpu.SemaphoreType.DMA,
            pltpu.SemaphoreType.DMA,
            pltpu.SemaphoreType.DMA,
        ],
        compiler_params=pltpu.CompilerParams(needs_layout_passes=False),
        interpret=interpret,
    )


def kernel(input):
    g = _noise()
    flat = _build()(input.reshape(-1), g)
    return flat.reshape(R, C)
